# baseline (speedup 1.0000x reference)
import functools

import jax
import jax.numpy as jnp
from jax.experimental import pallas as pl
from jax.experimental.pallas import tpu as pltpu

_K = 32
_S = 128
_PAD_VAL = 1.0e6


_T = 10
_BIG = 0x7FFFFFF


def _erode_kernel(xb_ref, xft_ref, xfr_ref, w0_ref, w1_ref, out_ref, dscr,
                  wv_scr, wc_scr, *, R, Np, C, K):
    G = Np // _S
    xb = xb_ref[...]
    xft = xft_ref[...]
    sqb = jnp.sum(xb * xb, axis=1, keepdims=True)
    sqf = jnp.sum(xft * xft, axis=0, keepdims=True)
    d = sqb + sqf - 2.0 * jnp.dot(xb, xft, preferred_element_type=jnp.float32)
    dscr[...] = d.reshape(R, G, _S)

    def extract(t, carry):
        prevv, prevc = carry
        cv = dscr[...]
        g3 = jax.lax.broadcasted_iota(jnp.int32, (R, G, _S), 1)
        p3 = jax.lax.broadcasted_iota(jnp.int32, (R, G, _S), 2)
        colv = g3 * _S + p3
        pv = prevv[:, :, None]
        pc = prevc[:, :, None]
        valid = (cv > pv) | ((cv == pv) & (colv > pc))
        mv = jnp.min(jnp.where(valid, cv, jnp.inf), axis=2, keepdims=True)
        mc = jnp.min(jnp.where(valid & (cv == mv), colv, _BIG), axis=2,
                     keepdims=True)
        mv = mv.reshape(R, G)
        mc = mc.reshape(R, G)
        wv_scr[:, pl.ds(t, 1), :] = mv[:, None, :]
        wc_scr[:, pl.ds(t, 1), :] = mc[:, None, :]
        return mv, mc

    jax.lax.fori_loop(
        0, _T, extract,
        (jnp.full((R, G), -jnp.inf, jnp.float32),
         jnp.full((R, G), -1, jnp.int32)))
    Wv = [wv_scr[:, t, :] for t in range(_T)]
    Wc = [wc_scr[:, t, :] for t in range(_T)]

    giota = jax.lax.broadcasted_iota(jnp.int32, (R, G), 1)
    piota = jax.lax.broadcasted_iota(jnp.int32, (R, 1, _S), 2)

    def body(j, carry):
        hp, headv, headc, acc0, acc1 = carry
        m = jnp.min(headv, axis=1, keepdims=True)
        amin = jnp.min(jnp.where(headv == m, headc, _BIG), axis=1,
                       keepdims=True)
        win = (headv == m) & (headc == amin)
        hp = hp + win.astype(jnp.int32)
        newv = jnp.full((R, G), jnp.inf, jnp.float32)
        newc = jnp.full((R, G), _BIG, jnp.int32)
        for t in range(_T - 1, -1, -1):
            sel = hp == t
            newv = jnp.where(sel, Wv[t], newv)
            newc = jnp.where(sel, Wc[t], newc)
        headv = jnp.where(win, newv, headv)
        headc = jnp.where(win, newc, headc)
        blk = amin // _S
        pos = amin - blk * _S
        og = (giota == blk).astype(jnp.float32)
        mid = jnp.dot(og, xfr_ref[...],
                      preferred_element_type=jnp.float32)
        mid3 = mid.reshape(R, C, _S)
        osel = piota == pos[:, :, None]
        neigh = jnp.sum(jnp.where(osel, mid3, 0.0), axis=2)
        w0j = w0_ref[pl.ds(j, 1), :]
        w1j = w1_ref[pl.ds(j, 1), :]
        acc0 = jnp.minimum(acc0, neigh - w0j)
        acc1 = jnp.minimum(acc1, neigh - w1j)
        return hp, headv, headc, acc0, acc1

    init = (jnp.zeros((R, G), jnp.int32), Wv[0], Wc[0],
            jnp.full((R, C), jnp.inf, jnp.float32),
            jnp.full((R, C), jnp.inf, jnp.float32))
    _, _, _, acc0, acc1 = jax.lax.fori_loop(0, K, body, init)
    out_ref[...] = jnp.concatenate([acc0, acc1], axis=1)


def _erode_layer(x_pad, w, *, R=128):
    Np, C = x_pad.shape
    K = w.shape[0]
    G = Np // _S
    xft = x_pad.T
    xfr = x_pad.reshape(G, _S, C).transpose(0, 2, 1).reshape(G, C * _S)
    w0 = w[:, :, 0]
    w1 = w[:, :, 1]
    out = pl.pallas_call(
        functools.partial(_erode_kernel, R=R, Np=Np, C=C, K=K),
        grid=(Np // R,),
        in_specs=[
            pl.BlockSpec((R, C), lambda i: (i, 0)),
            pl.BlockSpec((C, Np), lambda i: (0, 0)),
            pl.BlockSpec((G, C * _S), lambda i: (0, 0)),
            pl.BlockSpec((K, C), lambda i: (0, 0)),
            pl.BlockSpec((K, C), lambda i: (0, 0)),
        ],
        out_specs=pl.BlockSpec((R, 2 * C), lambda i: (i, 0)),
        out_shape=jax.ShapeDtypeStruct((Np, 2 * C), jnp.float32),
        scratch_shapes=[pltpu.VMEM((R, Np // _S, _S), jnp.float32),
                        pltpu.VMEM((R, _T, Np // _S), jnp.float32),
                        pltpu.VMEM((R, _T, Np // _S), jnp.int32)],
    )(x_pad[:, :C], xft, xfr, w0, w1)
    return out.reshape(Np, 2, C).transpose(0, 2, 1).reshape(Np, 2 * C)


def _mlp_kernel(h_ref, w1_ref, b1_ref, w2_ref, b2_ref, w3_ref, b3_ref,
                w4_ref, b4_ref, out_ref):
    h = h_ref[...]
    z = jnp.maximum(jnp.dot(h, w1_ref[...], preferred_element_type=jnp.float32)
                    + b1_ref[...], 0.0)
    z = jnp.maximum(jnp.dot(z, w2_ref[...], preferred_element_type=jnp.float32)
                    + b2_ref[...], 0.0)
    z = jnp.maximum(jnp.dot(z, w3_ref[...], preferred_element_type=jnp.float32)
                    + b3_ref[...], 0.0)
    logits = jnp.dot(z, w4_ref[...], preferred_element_type=jnp.float32) \
        + b4_ref[...]
    mx = jnp.max(logits, axis=1, keepdims=True)
    sh = logits - mx
    lse = jnp.log(jnp.sum(jnp.exp(sh), axis=1, keepdims=True))
    out_ref[...] = sh - lse


def _mlp_head(h, W_lin1, b_lin1, W_m1, b_m1, W_m2, b_m2, W_out, b_out, *,
              RM=512):
    Np, Cin = h.shape
    dims = [W_lin1.shape[1], W_m1.shape[1], W_m2.shape[1], W_out.shape[1]]
    return pl.pallas_call(
        _mlp_kernel,
        grid=(Np // RM,),
        in_specs=[
            pl.BlockSpec((RM, Cin), lambda i: (i, 0)),
            pl.BlockSpec(W_lin1.shape, lambda i: (0, 0)),
            pl.BlockSpec((1, dims[0]), lambda i: (0, 0)),
            pl.BlockSpec(W_m1.shape, lambda i: (0, 0)),
            pl.BlockSpec((1, dims[1]), lambda i: (0, 0)),
            pl.BlockSpec(W_m2.shape, lambda i: (0, 0)),
            pl.BlockSpec((1, dims[2]), lambda i: (0, 0)),
            pl.BlockSpec(W_out.shape, lambda i: (0, 0)),
            pl.BlockSpec((1, dims[3]), lambda i: (0, 0)),
        ],
        out_specs=pl.BlockSpec((RM, dims[3]), lambda i: (i, 0)),
        out_shape=jax.ShapeDtypeStruct((Np, dims[3]), jnp.float32),
    )(h, W_lin1, b_lin1[None, :], W_m1, b_m1[None, :], W_m2, b_m2[None, :],
      W_out, b_out[None, :])


def kernel(x, w1, w2, w3, W_lin1, b_lin1, W_m1, b_m1, W_m2, b_m2, W_out,
           b_out):
    N = x.shape[0]
    Np = ((N + 1279) // 1280) * 1280
    x_pad = jnp.pad(x, ((0, Np - N), (0, 0)), constant_values=_PAD_VAL)
    x1 = _erode_layer(x_pad, w1)
    x2 = _erode_layer(x1, w2)
    x3 = _erode_layer(x2, w3)
    h = jnp.concatenate([x1, x2, x3], axis=1)
    out = _mlp_head(h, W_lin1, b_lin1, W_m1, b_m1, W_m2, b_m2, W_out, b_out)
    return out[:N]


# speedup vs baseline: 1.8440x; 1.8440x over previous
import functools

import jax
import jax.numpy as jnp
from jax.experimental import pallas as pl
from jax.experimental.pallas import tpu as pltpu

_K = 32
_S = 128
_PAD_VAL = 1.0e6


def _erode_kernel(xb_ref, xft_ref, xfr_ref, w0_ref, w1_ref, out_ref, dscr,
                  *, R, Np, C, K):
    G = Np // _S
    xb = xb_ref[...]
    xft = xft_ref[...]
    sqb = jnp.sum(xb * xb, axis=1, keepdims=True)
    sqf = jnp.sum(xft * xft, axis=0, keepdims=True)
    d = sqb + sqf - 2.0 * jnp.dot(xb, xft, preferred_element_type=jnp.float32)
    dscr[...] = d

    iota = jax.lax.broadcasted_iota(jnp.int32, (R, Np), 1)
    giota = jax.lax.broadcasted_iota(jnp.int32, (R, G), 1)
    siota = jax.lax.broadcasted_iota(jnp.int32, (R, _S), 1)

    def body(j, carry):
        acc0, acc1 = carry
        dcur = dscr[...]
        m = jnp.min(dcur, axis=1, keepdims=True)
        amin = jnp.min(jnp.where(dcur == m, iota, Np), axis=1,
                       keepdims=True)
        onehot = iota == amin
        dscr[...] = jnp.where(onehot, jnp.inf, dcur)
        g = amin // _S
        s = amin - g * _S
        og = (giota == g).astype(jnp.float32)
        mid = jnp.dot(og, xfr_ref[...],
                      preferred_element_type=jnp.float32)
        mid3 = mid.reshape(R, C, _S)
        osel = (siota == s)[:, None, :]
        neigh = jnp.sum(jnp.where(osel, mid3, 0.0), axis=2)
        w0j = w0_ref[pl.ds(j, 1), :]
        w1j = w1_ref[pl.ds(j, 1), :]
        acc0 = jnp.minimum(acc0, neigh - w0j)
        acc1 = jnp.minimum(acc1, neigh - w1j)
        return acc0, acc1

    init = (jnp.full((R, C), jnp.inf, jnp.float32),
            jnp.full((R, C), jnp.inf, jnp.float32))
    acc0, acc1 = jax.lax.fori_loop(0, K, body, init)
    out_ref[...] = jnp.concatenate([acc0, acc1], axis=1)


def _erode_layer(x_pad, w, *, R=256):
    Np, C = x_pad.shape
    K = w.shape[0]
    G = Np // _S
    xft = x_pad.T
    xfr = x_pad.reshape(G, _S, C).transpose(0, 2, 1).reshape(G, C * _S)
    w0 = w[:, :, 0]
    w1 = w[:, :, 1]
    out = pl.pallas_call(
        functools.partial(_erode_kernel, R=R, Np=Np, C=C, K=K),
        grid=(Np // R,),
        in_specs=[
            pl.BlockSpec((R, C), lambda i: (i, 0)),
            pl.BlockSpec((C, Np), lambda i: (0, 0)),
            pl.BlockSpec((G, C * _S), lambda i: (0, 0)),
            pl.BlockSpec((K, C), lambda i: (0, 0)),
            pl.BlockSpec((K, C), lambda i: (0, 0)),
        ],
        out_specs=pl.BlockSpec((R, 2 * C), lambda i: (i, 0)),
        out_shape=jax.ShapeDtypeStruct((Np, 2 * C), jnp.float32),
        scratch_shapes=[pltpu.VMEM((R, Np), jnp.float32)],
    )(x_pad, xft, xfr, w0, w1)
    return out.reshape(Np, 2, C).transpose(0, 2, 1).reshape(Np, 2 * C)


def _mlp_kernel(h_ref, w1_ref, b1_ref, w2_ref, b2_ref, w3_ref, b3_ref,
                w4_ref, b4_ref, out_ref):
    h = h_ref[...]
    z = jnp.maximum(jnp.dot(h, w1_ref[...], preferred_element_type=jnp.float32)
                    + b1_ref[...], 0.0)
    z = jnp.maximum(jnp.dot(z, w2_ref[...], preferred_element_type=jnp.float32)
                    + b2_ref[...], 0.0)
    z = jnp.maximum(jnp.dot(z, w3_ref[...], preferred_element_type=jnp.float32)
                    + b3_ref[...], 0.0)
    logits = jnp.dot(z, w4_ref[...], preferred_element_type=jnp.float32) \
        + b4_ref[...]
    mx = jnp.max(logits, axis=1, keepdims=True)
    sh = logits - mx
    lse = jnp.log(jnp.sum(jnp.exp(sh), axis=1, keepdims=True))
    out_ref[...] = sh - lse


def _mlp_head(h, W_lin1, b_lin1, W_m1, b_m1, W_m2, b_m2, W_out, b_out, *,
              RM=512):
    Np, Cin = h.shape
    dims = [W_lin1.shape[1], W_m1.shape[1], W_m2.shape[1], W_out.shape[1]]
    return pl.pallas_call(
        _mlp_kernel,
        grid=(Np // RM,),
        in_specs=[
            pl.BlockSpec((RM, Cin), lambda i: (i, 0)),
            pl.BlockSpec(W_lin1.shape, lambda i: (0, 0)),
            pl.BlockSpec((1, dims[0]), lambda i: (0, 0)),
            pl.BlockSpec(W_m1.shape, lambda i: (0, 0)),
            pl.BlockSpec((1, dims[1]), lambda i: (0, 0)),
            pl.BlockSpec(W_m2.shape, lambda i: (0, 0)),
            pl.BlockSpec((1, dims[2]), lambda i: (0, 0)),
            pl.BlockSpec(W_out.shape, lambda i: (0, 0)),
            pl.BlockSpec((1, dims[3]), lambda i: (0, 0)),
        ],
        out_specs=pl.BlockSpec((RM, dims[3]), lambda i: (i, 0)),
        out_shape=jax.ShapeDtypeStruct((Np, dims[3]), jnp.float32),
    )(h, W_lin1, b_lin1[None, :], W_m1, b_m1[None, :], W_m2, b_m2[None, :],
      W_out, b_out[None, :])


def kernel(x, w1, w2, w3, W_lin1, b_lin1, W_m1, b_m1, W_m2, b_m2, W_out,
           b_out):
    N = x.shape[0]
    Np = ((N + 1279) // 1280) * 1280
    x_pad = jnp.pad(x, ((0, Np - N), (0, 0)), constant_values=_PAD_VAL)
    x1 = _erode_layer(x_pad, w1)
    x2 = _erode_layer(x1, w2)
    x3 = _erode_layer(x2, w3)
    h = jnp.concatenate([x1, x2, x3], axis=1)
    out = _mlp_head(h, W_lin1, b_lin1, W_m1, b_m1, W_m2, b_m2, W_out, b_out)
    return out[:N]


# speedup vs baseline: 1.8841x; 1.0217x over previous
import functools

import jax
import jax.numpy as jnp
from jax.experimental import pallas as pl
from jax.experimental.pallas import tpu as pltpu

_K = 32
_S = 128
_PAD_VAL = 1.0e6


def _erode_kernel(xb_ref, xft_ref, xfr_ref, w0_ref, w1_ref, out_ref, dscr,
                  *, R, Np, C, K):
    G = Np // _S
    xb = xb_ref[...]
    xft = xft_ref[...]
    sqb = jnp.sum(xb * xb, axis=1, keepdims=True)
    sqf = jnp.sum(xft * xft, axis=0, keepdims=True)
    d = sqb + sqf - 2.0 * jnp.dot(xb, xft, preferred_element_type=jnp.float32)
    dscr[...] = d

    iota = jax.lax.broadcasted_iota(jnp.int32, (R, Np), 1)
    giota = jax.lax.broadcasted_iota(jnp.int32, (R, G), 1)
    siota = jax.lax.broadcasted_iota(jnp.int32, (R, _S), 1)

    def body(j, carry):
        acc0, acc1 = carry
        dcur = dscr[...]
        m = jnp.min(dcur, axis=1, keepdims=True)
        amin = jnp.min(jnp.where(dcur == m, iota, Np), axis=1,
                       keepdims=True)
        onehot = iota == amin
        dscr[...] = jnp.where(onehot, jnp.inf, dcur)
        g = amin // _S
        s = amin - g * _S
        og = (giota == g).astype(jnp.float32)
        mid = jnp.dot(og, xfr_ref[...],
                      preferred_element_type=jnp.float32)
        mid3 = mid.reshape(R, C, _S)
        osel = (siota == s)[:, None, :]
        neigh = jnp.sum(jnp.where(osel, mid3, 0.0), axis=2)
        w0j = w0_ref[pl.ds(j, 1), :]
        w1j = w1_ref[pl.ds(j, 1), :]
        acc0 = jnp.minimum(acc0, neigh - w0j)
        acc1 = jnp.minimum(acc1, neigh - w1j)
        return acc0, acc1

    init = (jnp.full((R, C), jnp.inf, jnp.float32),
            jnp.full((R, C), jnp.inf, jnp.float32))
    acc0, acc1 = jax.lax.fori_loop(0, K, body, init)
    out_ref[...] = jnp.concatenate([acc0, acc1], axis=1)


def _erode_layer(x_pad, w, *, R=512):
    Np, C = x_pad.shape
    K = w.shape[0]
    G = Np // _S
    xft = x_pad.T
    xfr = x_pad.reshape(G, _S, C).transpose(0, 2, 1).reshape(G, C * _S)
    w0 = w[:, :, 0]
    w1 = w[:, :, 1]
    out = pl.pallas_call(
        functools.partial(_erode_kernel, R=R, Np=Np, C=C, K=K),
        grid=(Np // R,),
        in_specs=[
            pl.BlockSpec((R, C), lambda i: (i, 0)),
            pl.BlockSpec((C, Np), lambda i: (0, 0)),
            pl.BlockSpec((G, C * _S), lambda i: (0, 0)),
            pl.BlockSpec((K, C), lambda i: (0, 0)),
            pl.BlockSpec((K, C), lambda i: (0, 0)),
        ],
        out_specs=pl.BlockSpec((R, 2 * C), lambda i: (i, 0)),
        out_shape=jax.ShapeDtypeStruct((Np, 2 * C), jnp.float32),
        scratch_shapes=[pltpu.VMEM((R, Np), jnp.float32)],
    )(x_pad, xft, xfr, w0, w1)
    return out.reshape(Np, 2, C).transpose(0, 2, 1).reshape(Np, 2 * C)


def _mlp_kernel(h_ref, w1_ref, b1_ref, w2_ref, b2_ref, w3_ref, b3_ref,
                w4_ref, b4_ref, out_ref):
    h = h_ref[...]
    z = jnp.maximum(jnp.dot(h, w1_ref[...], preferred_element_type=jnp.float32)
                    + b1_ref[...], 0.0)
    z = jnp.maximum(jnp.dot(z, w2_ref[...], preferred_element_type=jnp.float32)
                    + b2_ref[...], 0.0)
    z = jnp.maximum(jnp.dot(z, w3_ref[...], preferred_element_type=jnp.float32)
                    + b3_ref[...], 0.0)
    logits = jnp.dot(z, w4_ref[...], preferred_element_type=jnp.float32) \
        + b4_ref[...]
    mx = jnp.max(logits, axis=1, keepdims=True)
    sh = logits - mx
    lse = jnp.log(jnp.sum(jnp.exp(sh), axis=1, keepdims=True))
    out_ref[...] = sh - lse


def _mlp_head(h, W_lin1, b_lin1, W_m1, b_m1, W_m2, b_m2, W_out, b_out, *,
              RM=512):
    Np, Cin = h.shape
    dims = [W_lin1.shape[1], W_m1.shape[1], W_m2.shape[1], W_out.shape[1]]
    return pl.pallas_call(
        _mlp_kernel,
        grid=(Np // RM,),
        in_specs=[
            pl.BlockSpec((RM, Cin), lambda i: (i, 0)),
            pl.BlockSpec(W_lin1.shape, lambda i: (0, 0)),
            pl.BlockSpec((1, dims[0]), lambda i: (0, 0)),
            pl.BlockSpec(W_m1.shape, lambda i: (0, 0)),
            pl.BlockSpec((1, dims[1]), lambda i: (0, 0)),
            pl.BlockSpec(W_m2.shape, lambda i: (0, 0)),
            pl.BlockSpec((1, dims[2]), lambda i: (0, 0)),
            pl.BlockSpec(W_out.shape, lambda i: (0, 0)),
            pl.BlockSpec((1, dims[3]), lambda i: (0, 0)),
        ],
        out_specs=pl.BlockSpec((RM, dims[3]), lambda i: (i, 0)),
        out_shape=jax.ShapeDtypeStruct((Np, dims[3]), jnp.float32),
    )(h, W_lin1, b_lin1[None, :], W_m1, b_m1[None, :], W_m2, b_m2[None, :],
      W_out, b_out[None, :])


def kernel(x, w1, w2, w3, W_lin1, b_lin1, W_m1, b_m1, W_m2, b_m2, W_out,
           b_out):
    N = x.shape[0]
    Np = ((N + 1279) // 1280) * 1280
    x_pad = jnp.pad(x, ((0, Np - N), (0, 0)), constant_values=_PAD_VAL)
    x1 = _erode_layer(x_pad, w1)
    x2 = _erode_layer(x1, w2)
    x3 = _erode_layer(x2, w3)
    h = jnp.concatenate([x1, x2, x3], axis=1)
    out = _mlp_head(h, W_lin1, b_lin1, W_m1, b_m1, W_m2, b_m2, W_out, b_out)
    return out[:N]


# speedup vs baseline: 1.9006x; 1.0088x over previous
import functools

import jax
import jax.numpy as jnp
from jax.experimental import pallas as pl
from jax.experimental.pallas import tpu as pltpu

_K = 32
_S = 128
_PAD_VAL = 1.0e6


def _erode_kernel(xb_ref, xft_ref, xfr_ref, w0_ref, w1_ref, out_ref, dscr,
                  *, R, Np, C, K):
    G = Np // _S
    xb = xb_ref[...]
    xft = xft_ref[...]
    sqb = jnp.sum(xb * xb, axis=1, keepdims=True)
    sqf = jnp.sum(xft * xft, axis=0, keepdims=True)
    d = sqb + sqf - 2.0 * jnp.dot(xb, xft, preferred_element_type=jnp.float32)
    dscr[...] = d

    iota = jax.lax.broadcasted_iota(jnp.int32, (R, Np), 1)
    giota = jax.lax.broadcasted_iota(jnp.int32, (R, G), 1)
    siota = jax.lax.broadcasted_iota(jnp.int32, (R, _S), 1)

    def body(j, carry):
        acc0, acc1 = carry
        dcur = dscr[...]
        m = jnp.min(dcur, axis=1, keepdims=True)
        amin = jnp.min(jnp.where(dcur == m, iota, Np), axis=1,
                       keepdims=True)
        onehot = iota == amin
        dscr[...] = jnp.where(onehot, jnp.inf, dcur)
        g = amin // _S
        s = amin - g * _S
        og = (giota == g).astype(jnp.float32)
        mid = jnp.dot(og, xfr_ref[...],
                      preferred_element_type=jnp.float32)
        mid3 = mid.reshape(R, C, _S)
        osel = (siota == s)[:, None, :]
        neigh = jnp.sum(jnp.where(osel, mid3, 0.0), axis=2)
        w0j = w0_ref[pl.ds(j, 1), :]
        w1j = w1_ref[pl.ds(j, 1), :]
        acc0 = jnp.minimum(acc0, neigh - w0j)
        acc1 = jnp.minimum(acc1, neigh - w1j)
        return acc0, acc1

    init = (jnp.full((R, C), jnp.inf, jnp.float32),
            jnp.full((R, C), jnp.inf, jnp.float32))
    acc0, acc1 = jax.lax.fori_loop(0, K, body, init)
    out_ref[...] = jnp.concatenate([acc0, acc1], axis=1)


def _erode_layer(x_pad, w, *, R=640):
    Np, C = x_pad.shape
    K = w.shape[0]
    G = Np // _S
    xft = x_pad.T
    xfr = x_pad.reshape(G, _S, C).transpose(0, 2, 1).reshape(G, C * _S)
    w0 = w[:, :, 0]
    w1 = w[:, :, 1]
    out = pl.pallas_call(
        functools.partial(_erode_kernel, R=R, Np=Np, C=C, K=K),
        grid=(Np // R,),
        in_specs=[
            pl.BlockSpec((R, C), lambda i: (i, 0)),
            pl.BlockSpec((C, Np), lambda i: (0, 0)),
            pl.BlockSpec((G, C * _S), lambda i: (0, 0)),
            pl.BlockSpec((K, C), lambda i: (0, 0)),
            pl.BlockSpec((K, C), lambda i: (0, 0)),
        ],
        out_specs=pl.BlockSpec((R, 2 * C), lambda i: (i, 0)),
        out_shape=jax.ShapeDtypeStruct((Np, 2 * C), jnp.float32),
        scratch_shapes=[pltpu.VMEM((R, Np), jnp.float32)],
    )(x_pad, xft, xfr, w0, w1)
    return out.reshape(Np, 2, C).transpose(0, 2, 1).reshape(Np, 2 * C)


def _mlp_kernel(h_ref, w1_ref, b1_ref, w2_ref, b2_ref, w3_ref, b3_ref,
                w4_ref, b4_ref, out_ref):
    h = h_ref[...]
    z = jnp.maximum(jnp.dot(h, w1_ref[...], preferred_element_type=jnp.float32)
                    + b1_ref[...], 0.0)
    z = jnp.maximum(jnp.dot(z, w2_ref[...], preferred_element_type=jnp.float32)
                    + b2_ref[...], 0.0)
    z = jnp.maximum(jnp.dot(z, w3_ref[...], preferred_element_type=jnp.float32)
                    + b3_ref[...], 0.0)
    logits = jnp.dot(z, w4_ref[...], preferred_element_type=jnp.float32) \
        + b4_ref[...]
    mx = jnp.max(logits, axis=1, keepdims=True)
    sh = logits - mx
    lse = jnp.log(jnp.sum(jnp.exp(sh), axis=1, keepdims=True))
    out_ref[...] = sh - lse


def _mlp_head(h, W_lin1, b_lin1, W_m1, b_m1, W_m2, b_m2, W_out, b_out, *,
              RM=512):
    Np, Cin = h.shape
    dims = [W_lin1.shape[1], W_m1.shape[1], W_m2.shape[1], W_out.shape[1]]
    return pl.pallas_call(
        _mlp_kernel,
        grid=(Np // RM,),
        in_specs=[
            pl.BlockSpec((RM, Cin), lambda i: (i, 0)),
            pl.BlockSpec(W_lin1.shape, lambda i: (0, 0)),
            pl.BlockSpec((1, dims[0]), lambda i: (0, 0)),
            pl.BlockSpec(W_m1.shape, lambda i: (0, 0)),
            pl.BlockSpec((1, dims[1]), lambda i: (0, 0)),
            pl.BlockSpec(W_m2.shape, lambda i: (0, 0)),
            pl.BlockSpec((1, dims[2]), lambda i: (0, 0)),
            pl.BlockSpec(W_out.shape, lambda i: (0, 0)),
            pl.BlockSpec((1, dims[3]), lambda i: (0, 0)),
        ],
        out_specs=pl.BlockSpec((RM, dims[3]), lambda i: (i, 0)),
        out_shape=jax.ShapeDtypeStruct((Np, dims[3]), jnp.float32),
    )(h, W_lin1, b_lin1[None, :], W_m1, b_m1[None, :], W_m2, b_m2[None, :],
      W_out, b_out[None, :])


def kernel(x, w1, w2, w3, W_lin1, b_lin1, W_m1, b_m1, W_m2, b_m2, W_out,
           b_out):
    N = x.shape[0]
    Np = ((N + 1279) // 1280) * 1280
    x_pad = jnp.pad(x, ((0, Np - N), (0, 0)), constant_values=_PAD_VAL)
    x1 = _erode_layer(x_pad, w1)
    x2 = _erode_layer(x1, w2)
    x3 = _erode_layer(x2, w3)
    h = jnp.concatenate([x1, x2, x3], axis=1)
    out = _mlp_head(h, W_lin1, b_lin1, W_m1, b_m1, W_m2, b_m2, W_out, b_out)
    return out[:N]
